# baseline (speedup 1.0000x reference)
import jax
import jax.numpy as jnp
from jax import lax
from jax.experimental import pallas as pl
from jax.experimental.pallas import tpu as pltpu
from jax.experimental.pallas import tpu_sc as plsc

BATCH = 16384
EMB = 64
NUM_WORKERS = 32
B_PER_W = BATCH // NUM_WORKERS
CHUNK = 128
N_CHUNKS = B_PER_W // CHUNK


def _gather_kernel(uids_hbm, mids_hbm, uemb_hbm, memb_hbm,
                   u_out, m_out, uidx_v, midx_v, urows_v, mrows_v, sem):
    wid = lax.axis_index("s") * 2 + lax.axis_index("c")
    base = wid * B_PER_W
    pltpu.sync_copy(uids_hbm.at[pl.ds(base, B_PER_W)], uidx_v)
    pltpu.sync_copy(mids_hbm.at[pl.ds(base, B_PER_W)], midx_v)
    copies = []
    for c in range(N_CHUNKS):
        sl = pl.ds(c * CHUNK, CHUNK)
        copies.append(pltpu.async_copy(
            uemb_hbm.at[uidx_v.at[sl]], urows_v.at[sl], sem))
        copies.append(pltpu.async_copy(
            memb_hbm.at[midx_v.at[sl]], mrows_v.at[sl], sem))
    for cp in copies:
        cp.wait()
    pltpu.sync_copy(urows_v, u_out.at[pl.ds(base, B_PER_W)])
    pltpu.sync_copy(mrows_v, m_out.at[pl.ds(base, B_PER_W)])


def _sc_gather(user_ids, movie_ids, user_emb, movie_emb):
    mesh = plsc.VectorSubcoreMesh(core_axis_name="c", subcore_axis_name="s")
    row_t = jax.ShapeDtypeStruct((BATCH, EMB), jnp.bfloat16)
    k = pl.kernel(
        _gather_kernel,
        out_type=(row_t, row_t),
        mesh=mesh,
        compiler_params=pltpu.CompilerParams(use_tc_tiling_on_sc=False),
        scratch_types=[
            pltpu.VMEM((B_PER_W,), jnp.int32),
            pltpu.VMEM((B_PER_W,), jnp.int32),
            pltpu.VMEM((B_PER_W, EMB), jnp.bfloat16),
            pltpu.VMEM((B_PER_W, EMB), jnp.bfloat16),
            pltpu.SemaphoreType.DMA,
        ],
    )
    return k(user_ids, movie_ids, user_emb, movie_emb)


def _mlp_kernel(u_ref, m_ref, w1u_ref, w1m_ref, b1_ref, w2_ref, b2_ref,
                w3_ref, b3_ref, out_ref):
    u = u_ref[...]
    m = m_ref[...]
    h1 = jnp.dot(u, w1u_ref[...], preferred_element_type=jnp.float32)
    h1 += jnp.dot(m, w1m_ref[...], preferred_element_type=jnp.float32)
    h1 = jnp.maximum(h1 + b1_ref[...], 0.0)
    h2 = jnp.dot(h1, w2_ref[...], preferred_element_type=jnp.float32)
    h2 = jnp.maximum(h2 + b2_ref[...], 0.0)
    logit = jnp.dot(h2, w3_ref[...], preferred_element_type=jnp.float32)
    out_ref[...] = jax.nn.sigmoid(logit + b3_ref[...])


def _tc_mlp(u_rows, m_rows, W1, b1, W2, b2, W3, b3):
    blk = 2048
    grid = (BATCH // blk,)
    w1u = W1[:, :EMB].T.astype(jnp.bfloat16)
    w1m = W1[:, EMB:].T.astype(jnp.bfloat16)
    w2 = W2.T
    w3 = W3.T
    b1r = b1.reshape(1, -1)
    b2r = b2.reshape(1, -1)
    b3r = b3.reshape(1, 1)
    out = pl.pallas_call(
        _mlp_kernel,
        grid=grid,
        in_specs=[
            pl.BlockSpec((blk, EMB), lambda i: (i, 0)),
            pl.BlockSpec((blk, EMB), lambda i: (i, 0)),
            pl.BlockSpec(w1u.shape, lambda i: (0, 0)),
            pl.BlockSpec(w1m.shape, lambda i: (0, 0)),
            pl.BlockSpec(b1r.shape, lambda i: (0, 0)),
            pl.BlockSpec(w2.shape, lambda i: (0, 0)),
            pl.BlockSpec(b2r.shape, lambda i: (0, 0)),
            pl.BlockSpec(w3.shape, lambda i: (0, 0)),
            pl.BlockSpec(b3r.shape, lambda i: (0, 0)),
        ],
        out_specs=pl.BlockSpec((blk, 1), lambda i: (i, 0)),
        out_shape=jax.ShapeDtypeStruct((BATCH, 1), jnp.float32),
    )(u_rows, m_rows, w1u, w1m, b1r, w2, b2r, w3, b3r)
    return out.reshape(BATCH)


@jax.jit
def kernel(user_ids, movie_ids, user_emb, movie_emb, W1, b1, W2, b2, W3, b3):
    u_rows, m_rows = _sc_gather(user_ids.astype(jnp.int32),
                                movie_ids.astype(jnp.int32),
                                user_emb.astype(jnp.bfloat16),
                                movie_emb.astype(jnp.bfloat16))
    return _tc_mlp(u_rows, m_rows, W1, b1, W2, b2, W3, b3)


# speedup vs baseline: 2.0150x; 2.0150x over previous
import dataclasses

import jax
import jax.numpy as jnp
from jax import lax
from jax.experimental import pallas as pl
from jax.experimental.pallas import tpu as pltpu
from jax.experimental.pallas import tpu_sc as plsc

BATCH = 16384
EMB = 64
V = 1_000_000
NUM_WORKERS = 32
CW = 512
NFULL = V // CW
TAIL_START = NFULL * CW
TAIL_W = V - TAIL_START
CPT = NFULL // NUM_WORKERS
CAP = 32
NB_MAX = CPT + 2
STAGE = 128
DUMP = BATCH


def _refill_sidx(sidx):
    for l in range(0, STAGE, 16):
        sidx[0, pl.ds(l, 16)] = jnp.full((16,), DUMP, jnp.int32)


def _flush(stage, sidx, out_hbm, cnt_s):
    pltpu.sync_copy(stage, out_hbm.at[sidx.at[0]])
    _refill_sidx(sidx)
    cnt_s[0] = 0


def _claim_round(pending, jc, idv, pos, lane, bids, bpos, counts, claim):
    plsc.store_scatter(claim, [jc], lane, mask=pending)
    seen = plsc.load_gather(claim, [jc], mask=pending)
    won = (seen == lane) & pending
    base = plsc.load_gather(counts, [jc], mask=won)
    slot = jc * CAP + jnp.clip(base, 0, CAP - 1)
    plsc.store_scatter(bids, [slot], idv, mask=won)
    plsc.store_scatter(bpos, [slot], pos, mask=won)
    plsc.store_scatter(counts, [jc], base + 1, mask=won)
    return pending & ~won


def _extract(idvec, posvec, valid, buf, width, s0, stage, sidx, cnt_s):
    col = jnp.clip(idvec - s0, 0, width - 1)
    pref = plsc.cumsum(jnp.ones((16,), jnp.int32), mask=valid)
    cnt = cnt_s[0]
    slot = jnp.clip(cnt + pref - 1, 0, STAGE - 1)
    for s in range(EMB):
        row = jnp.full((16,), s, jnp.int32)
        vals = plsc.load_gather(buf, [row, col], mask=valid)
        plsc.store_scatter(stage, [slot, row], vals, mask=valid)
    plsc.store_scatter(sidx, [jnp.zeros((16,), jnp.int32), slot], posvec,
                       mask=valid)
    k = plsc.all_reduce_population_count(valid)[0]
    cnt_s[0] = cnt + k


def _process_bucket(i, buf, width, s0, lane, bids, bpos, counts, stage, sidx,
                    out_hbm, cnt_s):
    jv = jnp.full((16,), i, jnp.int32)
    kv = plsc.load_gather(counts, [jv])
    b_ids = bids[pl.ds(i * CAP, 16)]
    b_pos = bpos[pl.ds(i * CAP, 16)]
    valid = lane < kv
    _extract(b_ids, b_pos, valid, buf, width, s0, stage, sidx, cnt_s)

    @pl.when(cnt_s[0] >= STAGE - 16)
    def _():
        _flush(stage, sidx, out_hbm, cnt_s)

    @pl.when(kv[0] > 16)
    def _():
        b_ids2 = bids[pl.ds(i * CAP + 16, 16)]
        b_pos2 = bpos[pl.ds(i * CAP + 16, 16)]
        valid2 = (lane + 16) < kv
        _extract(b_ids2, b_pos2, valid2, buf, width, s0, stage, sidx, cnt_s)

        @pl.when(cnt_s[0] >= STAGE - 16)
        def _():
            _flush(stage, sidx, out_hbm, cnt_s)


def _sweep_kernel(uids_hbm, mids_hbm, utv_hbm, mtv_hbm, u_out, m_out,
                  ids_v, bufA, bufB, tailbuf, stage, sidx, bids, bpos,
                  counts, claim, cnt_s, semA, semB):
    wid = lax.axis_index("s") * 2 + lax.axis_index("c")
    lane = lax.iota(jnp.int32, 16)
    is31 = (wid == NUM_WORKERS - 1).astype(jnp.int32)
    lo = wid * CPT
    nfull = CPT + is31
    nb = CPT + 2 * is31
    _refill_sidx(sidx)

    for ids_hbm, tv_hbm, out_hbm in ((uids_hbm, utv_hbm, u_out),
                                     (mids_hbm, mtv_hbm, m_out)):
        pltpu.sync_copy(ids_hbm, ids_v)
        for l in range(0, 64, 16):
            counts[pl.ds(l, 16)] = jnp.zeros((16,), jnp.int32)
        cnt_s[0] = 0

        @pl.loop(0, BATCH, step=16)
        def _(i):
            idv = ids_v[pl.ds(i, 16)]
            c = lax.shift_right_logical(idv, 9)
            j = c - lo
            inr = (j >= 0) & (j < nb)

            @pl.when(plsc.all_reduce_population_count(inr)[0] > 0)
            def _():
                jc = jnp.clip(j, 0, NB_MAX - 1)
                pos = lane + i
                p = _claim_round(inr, jc, idv, pos, lane, bids, bpos,
                                 counts, claim)

                @pl.when(plsc.all_reduce_population_count(p)[0] > 0)
                def _():
                    q = p
                    for _r in range(5):
                        q = _claim_round(q, jc, idv, pos, lane, bids, bpos,
                                         counts, claim)

        def fire(buf, c, sem):
            s = (lo + c) * CW
            pltpu.async_copy(tv_hbm.at[:, pl.ds(s, CW)], buf, sem)

        def drain(buf, sem):
            pltpu.make_async_copy(tv_hbm.at[:, pl.ds(0, CW)], buf, sem).wait()

        fire(bufA, 0, semA)

        @pl.when(nfull > 1)
        def _():
            fire(bufB, 1, semB)

        @pl.loop(0, CPT + 1, step=2)
        def _(i):
            @pl.when(i < nfull)
            def _():
                drain(bufA, semA)
                _process_bucket(i, bufA, CW, (lo + i) * CW, lane, bids, bpos,
                                counts, stage, sidx, out_hbm, cnt_s)

                @pl.when(i + 2 < nfull)
                def _():
                    fire(bufA, i + 2, semA)

            @pl.when(i + 1 < nfull)
            def _():
                drain(bufB, semB)
                _process_bucket(i + 1, bufB, CW, (lo + i + 1) * CW, lane,
                                bids, bpos, counts, stage, sidx, out_hbm,
                                cnt_s)

                @pl.when(i + 3 < nfull)
                def _():
                    fire(bufB, i + 3, semB)

        @pl.when(is31 > 0)
        def _():
            pltpu.sync_copy(tv_hbm.at[:, pl.ds(TAIL_START, TAIL_W)], tailbuf)
            _process_bucket(CPT + 1, tailbuf, TAIL_W, TAIL_START, lane, bids,
                            bpos, counts, stage, sidx, out_hbm, cnt_s)

        @pl.when(cnt_s[0] > 0)
        def _():
            _flush(stage, sidx, out_hbm, cnt_s)


def _sc_gather(user_ids, movie_ids, user_emb, movie_emb):
    mesh = plsc.VectorSubcoreMesh(core_axis_name="c", subcore_axis_name="s")
    out_t = jax.ShapeDtypeStruct((BATCH + 1, 128), jnp.float32)
    k = pl.kernel(
        _sweep_kernel,
        out_type=(out_t, out_t),
        mesh=mesh,
        compiler_params=dataclasses.replace(
            pltpu.CompilerParams(use_tc_tiling_on_sc=True),
            needs_layout_passes=False),
        scratch_types=[
            pltpu.VMEM((BATCH,), jnp.int32),
            pltpu.VMEM((EMB, CW), jnp.float32),
            pltpu.VMEM((EMB, CW), jnp.float32),
            pltpu.VMEM((EMB, TAIL_W), jnp.float32),
            pltpu.VMEM((STAGE, 128), jnp.float32),
            pltpu.VMEM((1, STAGE), jnp.int32),
            pltpu.VMEM((NB_MAX * CAP,), jnp.int32),
            pltpu.VMEM((NB_MAX * CAP,), jnp.int32),
            pltpu.VMEM((64,), jnp.int32),
            pltpu.VMEM((64,), jnp.int32),
            pltpu.SMEM((8,), jnp.int32),
            pltpu.SemaphoreType.DMA,
            pltpu.SemaphoreType.DMA,
        ],
    )
    return k(user_ids, movie_ids, user_emb.T, movie_emb.T)


def _mlp_kernel(u_ref, m_ref, w1u_ref, w1m_ref, b1_ref, w2_ref, b2_ref,
                w3_ref, b3_ref, out_ref):
    u = u_ref[:, :EMB]
    m = m_ref[:, :EMB]
    h1 = jnp.dot(u, w1u_ref[...], preferred_element_type=jnp.float32)
    h1 += jnp.dot(m, w1m_ref[...], preferred_element_type=jnp.float32)
    h1 = jnp.maximum(h1 + b1_ref[...], 0.0)
    h2 = jnp.dot(h1, w2_ref[...], preferred_element_type=jnp.float32)
    h2 = jnp.maximum(h2 + b2_ref[...], 0.0)
    logit = jnp.dot(h2, w3_ref[...], preferred_element_type=jnp.float32)
    out_ref[...] = jax.nn.sigmoid(logit + b3_ref[...])


def _tc_mlp(u_rows, m_rows, W1, b1, W2, b2, W3, b3):
    blk = 2048
    grid = (BATCH // blk,)
    w1u = W1[:, :EMB].T
    w1m = W1[:, EMB:].T
    w2 = W2.T
    w3 = W3.T
    b1r = b1.reshape(1, -1)
    b2r = b2.reshape(1, -1)
    b3r = b3.reshape(1, 1)
    out = pl.pallas_call(
        _mlp_kernel,
        grid=grid,
        in_specs=[
            pl.BlockSpec((blk, 128), lambda i: (i, 0)),
            pl.BlockSpec((blk, 128), lambda i: (i, 0)),
            pl.BlockSpec(w1u.shape, lambda i: (0, 0)),
            pl.BlockSpec(w1m.shape, lambda i: (0, 0)),
            pl.BlockSpec(b1r.shape, lambda i: (0, 0)),
            pl.BlockSpec(w2.shape, lambda i: (0, 0)),
            pl.BlockSpec(b2r.shape, lambda i: (0, 0)),
            pl.BlockSpec(w3.shape, lambda i: (0, 0)),
            pl.BlockSpec(b3r.shape, lambda i: (0, 0)),
        ],
        out_specs=pl.BlockSpec((blk, 1), lambda i: (i, 0)),
        out_shape=jax.ShapeDtypeStruct((BATCH, 1), jnp.float32),
    )(u_rows, m_rows, w1u, w1m, b1r, w2, b2r, w3, b3r)
    return out.reshape(BATCH)


@jax.jit
def kernel(user_ids, movie_ids, user_emb, movie_emb, W1, b1, W2, b2, W3, b3):
    u_wide, m_wide = _sc_gather(user_ids.astype(jnp.int32),
                                movie_ids.astype(jnp.int32),
                                user_emb, movie_emb)
    return _tc_mlp(u_wide[:BATCH], m_wide[:BATCH], W1, b1, W2, b2, W3, b3)
